# baseline (speedup 1.0000x reference)
import functools

import jax
import jax.numpy as jnp
from jax import lax
from jax.experimental import pallas as pl
from jax.experimental.pallas import tpu as pltpu
from jax.experimental.pallas import tpu_sc as plsc

NUM_NODES = 1000000
MEMORY_DIM = 64
BATCH = 16384
NC = 2
NS = 16
NW = NC * NS
B_PER_W = BATCH // NW


def _sc_gather_body(table_hbm, idx_hbm, out_hbm, idx_v, rows_v, sem):
    wid = lax.axis_index("s") * NC + lax.axis_index("c")
    base = wid * B_PER_W
    pltpu.sync_copy(idx_hbm.at[pl.ds(base, B_PER_W)], idx_v)
    pltpu.async_copy(table_hbm.at[idx_v], rows_v, sem).wait()
    pltpu.sync_copy(rows_v, out_hbm.at[pl.ds(base, B_PER_W)])


@functools.cache
def _sc_gather():
    return pl.kernel(
        _sc_gather_body,
        out_type=jax.ShapeDtypeStruct((BATCH, MEMORY_DIM), jnp.float32),
        mesh=plsc.VectorSubcoreMesh(core_axis_name="c", subcore_axis_name="s"),
        scratch_types=[
            pltpu.VMEM((B_PER_W,), jnp.int32),
            pltpu.VMEM((B_PER_W, MEMORY_DIM), jnp.float32),
            pltpu.SemaphoreType.DMA,
        ],
        compiler_params=pltpu.CompilerParams(use_tc_tiling_on_sc=False),
    )


def _gru_body(h_ref, wr_ref, wz_ref, wn_ref, br_ref, bz_ref, bin_ref,
              bhn_ref, out_ref):
    h = h_ref[...]
    ghr = jnp.dot(h, wr_ref[...], preferred_element_type=jnp.float32)
    ghz = jnp.dot(h, wz_ref[...], preferred_element_type=jnp.float32)
    ghn = jnp.dot(h, wn_ref[...], preferred_element_type=jnp.float32)
    r = jax.nn.sigmoid(ghr + br_ref[...])
    z = jax.nn.sigmoid(ghz + bz_ref[...])
    n = jnp.tanh(bin_ref[...] + r * (ghn + bhn_ref[...]))
    out_ref[...] = (1.0 - z) * n + z * h


def _gru_call(h, wr, wz, wn, br, bz, bin_, bhn, *, interpret=False):
    BR = 2048
    grid = BATCH // BR
    row_spec = pl.BlockSpec((BR, MEMORY_DIM), lambda i: (i, 0))
    w_spec = pl.BlockSpec((MEMORY_DIM, MEMORY_DIM), lambda i: (0, 0))
    b_spec = pl.BlockSpec((1, MEMORY_DIM), lambda i: (0, 0))
    return pl.pallas_call(
        _gru_body,
        grid=(grid,),
        in_specs=[row_spec, w_spec, w_spec, w_spec,
                  b_spec, b_spec, b_spec, b_spec],
        out_specs=row_spec,
        out_shape=jax.ShapeDtypeStruct((BATCH, MEMORY_DIM), jnp.float32),
        interpret=interpret,
    )(h, wr, wz, wn, br, bz, bin_, bhn)


def kernel(n_id, memory, W_ih, W_hh, b_ih, b_hh):
    del W_ih
    H = MEMORY_DIM
    h = _sc_gather()(memory, n_id)
    wr = W_hh[:H].T
    wz = W_hh[H:2 * H].T
    wn = W_hh[2 * H:].T
    br = (b_ih[:H] + b_hh[:H]).reshape(1, H)
    bz = (b_ih[H:2 * H] + b_hh[H:2 * H]).reshape(1, H)
    bin_ = b_ih[2 * H:].reshape(1, H)
    bhn = b_hh[2 * H:].reshape(1, H)
    new_mem = _gru_call(h, wr, wz, wn, br, bz, bin_, bhn)
    last_update = jnp.zeros((n_id.shape[0],), dtype=jnp.int32)
    return (new_mem, last_update)


# speedup vs baseline: 1.7042x; 1.7042x over previous
import functools

import jax
import jax.numpy as jnp
from jax import lax
from jax.experimental import pallas as pl
from jax.experimental.pallas import tpu as pltpu
from jax.experimental.pallas import tpu_sc as plsc

NUM_NODES = 1000000
MEMORY_DIM = 64
BATCH = 16384
NC = 2
NS = 16
NW = NC * NS
B_PER_W = BATCH // NW


def _sc_gather_body(table_hbm, idx_hbm, out_hbm, idx_v, rows_v, sem):
    wid = lax.axis_index("s") * NC + lax.axis_index("c")
    base = wid * B_PER_W
    pltpu.sync_copy(idx_hbm.at[pl.ds(base, B_PER_W)], idx_v)

    def issue(c, carry):
        v = idx_v[pl.ds(c * 16, 16)]
        for j in range(16):
            r = v[j]
            pltpu.make_async_copy(
                table_hbm.at[pl.ds(r, 1)],
                rows_v.at[pl.ds(c * 16 + j, 1)],
                sem,
            ).start()
        return carry

    lax.fori_loop(0, B_PER_W // 16, issue, 0)
    pltpu.make_async_copy(
        table_hbm.at[pl.ds(0, B_PER_W)], rows_v, sem
    ).wait()
    pltpu.sync_copy(rows_v, out_hbm.at[pl.ds(base, B_PER_W)])


@functools.cache
def _sc_gather():
    return pl.kernel(
        _sc_gather_body,
        out_type=jax.ShapeDtypeStruct((BATCH, MEMORY_DIM), jnp.float32),
        mesh=plsc.VectorSubcoreMesh(core_axis_name="c", subcore_axis_name="s"),
        scratch_types=[
            pltpu.VMEM((B_PER_W,), jnp.int32),
            pltpu.VMEM((B_PER_W, MEMORY_DIM), jnp.float32),
            pltpu.SemaphoreType.DMA,
        ],
        compiler_params=pltpu.CompilerParams(use_tc_tiling_on_sc=True),
    )


def _gru_body(h_ref, wr_ref, wz_ref, wn_ref, br_ref, bz_ref, bin_ref,
              bhn_ref, out_ref):
    h = h_ref[...]
    ghr = jnp.dot(h, wr_ref[...], preferred_element_type=jnp.float32)
    ghz = jnp.dot(h, wz_ref[...], preferred_element_type=jnp.float32)
    ghn = jnp.dot(h, wn_ref[...], preferred_element_type=jnp.float32)
    r = jax.nn.sigmoid(ghr + br_ref[...])
    z = jax.nn.sigmoid(ghz + bz_ref[...])
    n = jnp.tanh(bin_ref[...] + r * (ghn + bhn_ref[...]))
    out_ref[...] = (1.0 - z) * n + z * h


def _gru_call(h, wr, wz, wn, br, bz, bin_, bhn, *, interpret=False):
    BR = 2048
    grid = BATCH // BR
    row_spec = pl.BlockSpec((BR, MEMORY_DIM), lambda i: (i, 0))
    w_spec = pl.BlockSpec((MEMORY_DIM, MEMORY_DIM), lambda i: (0, 0))
    b_spec = pl.BlockSpec((1, MEMORY_DIM), lambda i: (0, 0))
    return pl.pallas_call(
        _gru_body,
        grid=(grid,),
        in_specs=[row_spec, w_spec, w_spec, w_spec,
                  b_spec, b_spec, b_spec, b_spec],
        out_specs=row_spec,
        out_shape=jax.ShapeDtypeStruct((BATCH, MEMORY_DIM), jnp.float32),
        interpret=interpret,
    )(h, wr, wz, wn, br, bz, bin_, bhn)


def kernel(n_id, memory, W_ih, W_hh, b_ih, b_hh):
    del W_ih
    H = MEMORY_DIM
    h = _sc_gather()(memory, n_id)
    wr = W_hh[:H].T
    wz = W_hh[H:2 * H].T
    wn = W_hh[2 * H:].T
    br = (b_ih[:H] + b_hh[:H]).reshape(1, H)
    bz = (b_ih[H:2 * H] + b_hh[H:2 * H]).reshape(1, H)
    bin_ = b_ih[2 * H:].reshape(1, H)
    bhn = b_hh[2 * H:].reshape(1, H)
    new_mem = _gru_call(h, wr, wz, wn, br, bz, bin_, bhn)
    last_update = jnp.zeros((n_id.shape[0],), dtype=jnp.int32)
    return (new_mem, last_update)


# speedup vs baseline: 3.0792x; 1.8068x over previous
import functools

import jax
import jax.numpy as jnp
from jax import lax
from jax.experimental import pallas as pl
from jax.experimental.pallas import tpu as pltpu
from jax.experimental.pallas import tpu_sc as plsc

NUM_NODES = 1000000
MEMORY_DIM = 64
BATCH = 16384
NC = 2
NS = 16
NW = NC * NS
B_PER_W = BATCH // NW

BLK_W = 128
N_BLKS = (NUM_NODES + BLK_W - 1) // BLK_W
TAIL_BLK = N_BLKS - 1
TAIL_W = NUM_NODES - TAIL_BLK * BLK_W
NBUF = 2


def _sc_gather_body(memT, sid_hbm, spos_hbm, out_hbm,
                    ids_v, pos_v, blocks_v, rows_v, posb_v, dsem, ssem):
    wid = lax.axis_index("s") * NC + lax.axis_index("c")
    base = wid * B_PER_W
    pltpu.sync_copy(sid_hbm.at[pl.ds(base, B_PER_W)], ids_v)
    pltpu.sync_copy(spos_hbm.at[pl.ds(base, B_PER_W)], pos_v)

    def blk_copy(b):
        slot = lax.rem(b, NBUF)
        return pltpu.make_async_copy(
            memT.at[:, pl.ds(pl.multiple_of(b * BLK_W, BLK_W), BLK_W)],
            blocks_v.at[slot],
            dsem.at[slot],
        )

    first = ids_v[pl.ds(0, 16)]
    lastv = ids_v[pl.ds(B_PER_W - 16, 16)]
    first_blk = lax.shift_right_logical(first[0], 7)
    my_last_blk = lax.shift_right_logical(lastv[15], 7)
    blk_copy(first_blk).start()

    def step_body(w, carry):
        blk_copy(jnp.minimum(w + 1, my_last_blk)).start()
        blk_copy(w).wait()
        return carry

    def group_body(g, waited):
        def chunk_body(cc, waited):
            c = g * 8 + cc
            v_ids = ids_v[pl.ds(c * 16, 16)]
            v_pos = pos_v[pl.ds(c * 16, 16)]
            posb_v[0, pl.ds(cc * 16, 16)] = v_pos
            for j in range(16):
                id_s = v_ids[j]
                tgt = lax.shift_right_logical(id_s, 7)
                lax.fori_loop(waited + 1, tgt + 1, step_body, 0)
                waited = jnp.maximum(waited, tgt)

                slot = lax.rem(tgt, NBUF)
                col = lax.bitwise_and(id_s, 127)
                lane_idx = jnp.broadcast_to(col, (16,))
                blk_ref = blocks_v.at[slot]
                for ccc in range(4):
                    ridx = jnp.arange(16, dtype=jnp.int32) + (16 * ccc)
                    val = plsc.load_gather(blk_ref, [ridx, lane_idx])
                    rows_v[cc * 16 + j, pl.ds(16 * ccc, 16)] = val
            return waited

        waited = lax.fori_loop(0, 8, chunk_body, waited)
        pltpu.async_copy(rows_v, out_hbm.at[posb_v.at[0]], ssem).wait()
        return waited

    lax.fori_loop(0, B_PER_W // 128, group_body, first_blk - 1)
    blk_copy(my_last_blk).wait()


@functools.cache
def _sc_gather():
    return pl.kernel(
        _sc_gather_body,
        out_type=jax.ShapeDtypeStruct((BATCH, BLK_W), jnp.float32),
        mesh=plsc.VectorSubcoreMesh(core_axis_name="c", subcore_axis_name="s"),
        scratch_types=[
            pltpu.VMEM((B_PER_W,), jnp.int32),
            pltpu.VMEM((B_PER_W,), jnp.int32),
            pltpu.VMEM((NBUF, MEMORY_DIM, BLK_W), jnp.float32),
            pltpu.VMEM((128, BLK_W), jnp.float32),
            pltpu.VMEM((1, 128), jnp.int32),
            pltpu.SemaphoreType.DMA((NBUF,)),
            pltpu.SemaphoreType.DMA,
        ],
        compiler_params=pltpu.CompilerParams(use_tc_tiling_on_sc=True, needs_layout_passes=False),
    )


def _gru_body(h_ref, wr_ref, wz_ref, wn_ref, br_ref, bz_ref, bin_ref,
              bhn_ref, out_ref):
    h = h_ref[...][:, :MEMORY_DIM]
    ghr = jnp.dot(h, wr_ref[...], preferred_element_type=jnp.float32)
    ghz = jnp.dot(h, wz_ref[...], preferred_element_type=jnp.float32)
    ghn = jnp.dot(h, wn_ref[...], preferred_element_type=jnp.float32)
    r = jax.nn.sigmoid(ghr + br_ref[...])
    z = jax.nn.sigmoid(ghz + bz_ref[...])
    n = jnp.tanh(bin_ref[...] + r * (ghn + bhn_ref[...]))
    out_ref[...] = (1.0 - z) * n + z * h


def _gru_call(h128, wr, wz, wn, br, bz, bin_, bhn, *, interpret=False):
    BR = 2048
    grid = BATCH // BR
    in_spec = pl.BlockSpec((BR, BLK_W), lambda i: (i, 0))
    out_spec = pl.BlockSpec((BR, MEMORY_DIM), lambda i: (i, 0))
    w_spec = pl.BlockSpec((MEMORY_DIM, MEMORY_DIM), lambda i: (0, 0))
    b_spec = pl.BlockSpec((1, MEMORY_DIM), lambda i: (0, 0))
    return pl.pallas_call(
        _gru_body,
        grid=(grid,),
        in_specs=[in_spec, w_spec, w_spec, w_spec,
                  b_spec, b_spec, b_spec, b_spec],
        out_specs=out_spec,
        out_shape=jax.ShapeDtypeStruct((BATCH, MEMORY_DIM), jnp.float32),
        interpret=interpret,
    )(h128, wr, wz, wn, br, bz, bin_, bhn)


def kernel(n_id, memory, W_ih, W_hh, b_ih, b_hh):
    del W_ih
    H = MEMORY_DIM
    memT = memory.T
    sid, spos = lax.sort_key_val(n_id, jnp.arange(BATCH, dtype=jnp.int32))
    h128 = _sc_gather()(memT, sid, spos)
    wr = W_hh[:H].T
    wz = W_hh[H:2 * H].T
    wn = W_hh[2 * H:].T
    br = (b_ih[:H] + b_hh[:H]).reshape(1, H)
    bz = (b_ih[H:2 * H] + b_hh[H:2 * H]).reshape(1, H)
    bin_ = b_ih[2 * H:].reshape(1, H)
    bhn = b_hh[2 * H:].reshape(1, H)
    new_mem = _gru_call(h128, wr, wz, wn, br, bz, bin_, bhn)
    last_update = jnp.zeros((n_id.shape[0],), dtype=jnp.int32)
    return (new_mem, last_update)


# speedup vs baseline: 3.8848x; 1.2616x over previous
import functools

import jax
import jax.numpy as jnp
from jax import lax
from jax.experimental import pallas as pl
from jax.experimental.pallas import tpu as pltpu
from jax.experimental.pallas import tpu_sc as plsc

NUM_NODES = 1000000
MEMORY_DIM = 64
BATCH = 16384
NC = 2
NS = 16
NW = NC * NS
B_PER_W = BATCH // NW

BLK_W = 512
BLK_SHIFT = 9
PAD_LANES = ((NUM_NODES + 127) // 128) * 128
BASE_MAX = PAD_LANES - BLK_W
NBUF = 2


def _sc_gather_body(memT, sid_hbm, spos_hbm, out_hbm,
                    ids_v, pos_v, blocks_v, rows_v, posb_v, dsem, ssem):
    wid = lax.axis_index("s") * NC + lax.axis_index("c")
    base = wid * B_PER_W
    pltpu.sync_copy(sid_hbm.at[pl.ds(base, B_PER_W)], ids_v)
    pltpu.sync_copy(spos_hbm.at[pl.ds(base, B_PER_W)], pos_v)

    def blk_copy(b):
        slot = lax.rem(b, NBUF)
        base = jnp.minimum(b * BLK_W, BASE_MAX)
        return pltpu.make_async_copy(
            memT.at[:, pl.ds(pl.multiple_of(base, 128), BLK_W)],
            blocks_v.at[slot],
            dsem.at[slot],
        )

    first = ids_v[pl.ds(0, 16)]
    lastv = ids_v[pl.ds(B_PER_W - 16, 16)]
    first_blk = lax.shift_right_logical(first[0], BLK_SHIFT)
    my_last_blk = lax.shift_right_logical(lastv[15], BLK_SHIFT)
    blk_copy(first_blk).start()

    def step_body(w, carry):
        blk_copy(jnp.minimum(w + 1, my_last_blk)).start()
        blk_copy(w).wait()
        return carry

    def group_body(g, waited):
        def chunk_body(cc, waited):
            c = g * 8 + cc
            v_ids = ids_v[pl.ds(c * 16, 16)]
            v_pos = pos_v[pl.ds(c * 16, 16)]
            posb_v[0, pl.ds(cc * 16, 16)] = v_pos
            for j in range(16):
                id_s = v_ids[j]
                tgt = lax.shift_right_logical(id_s, BLK_SHIFT)
                lax.fori_loop(waited + 1, tgt + 1, step_body, 0)
                waited = jnp.maximum(waited, tgt)

                slot = lax.rem(tgt, NBUF)
                col = id_s - jnp.minimum(tgt * BLK_W, BASE_MAX)
                lane_idx = jnp.broadcast_to(col, (16,))
                blk_ref = blocks_v.at[slot]
                for ccc in range(4):
                    ridx = jnp.arange(16, dtype=jnp.int32) + (16 * ccc)
                    val = plsc.load_gather(blk_ref, [ridx, lane_idx])
                    rows_v[cc * 16 + j, pl.ds(16 * ccc, 16)] = val
            return waited

        waited = lax.fori_loop(0, 8, chunk_body, waited)
        pltpu.async_copy(rows_v, out_hbm.at[posb_v.at[0]], ssem).wait()
        return waited

    lax.fori_loop(0, B_PER_W // 128, group_body, first_blk - 1)
    blk_copy(my_last_blk).wait()


@functools.cache
def _sc_gather():
    return pl.kernel(
        _sc_gather_body,
        out_type=jax.ShapeDtypeStruct((BATCH, 128), jnp.float32),
        mesh=plsc.VectorSubcoreMesh(core_axis_name="c", subcore_axis_name="s"),
        scratch_types=[
            pltpu.VMEM((B_PER_W,), jnp.int32),
            pltpu.VMEM((B_PER_W,), jnp.int32),
            pltpu.VMEM((NBUF, MEMORY_DIM, BLK_W), jnp.float32),
            pltpu.VMEM((128, 128), jnp.float32),
            pltpu.VMEM((1, 128), jnp.int32),
            pltpu.SemaphoreType.DMA((NBUF,)),
            pltpu.SemaphoreType.DMA,
        ],
        compiler_params=pltpu.CompilerParams(use_tc_tiling_on_sc=True, needs_layout_passes=False),
    )


def _gru_body(h_ref, wr_ref, wz_ref, wn_ref, br_ref, bz_ref, bin_ref,
              bhn_ref, out_ref):
    h = h_ref[...][:, :MEMORY_DIM]
    ghr = jnp.dot(h, wr_ref[...], preferred_element_type=jnp.float32)
    ghz = jnp.dot(h, wz_ref[...], preferred_element_type=jnp.float32)
    ghn = jnp.dot(h, wn_ref[...], preferred_element_type=jnp.float32)
    r = jax.nn.sigmoid(ghr + br_ref[...])
    z = jax.nn.sigmoid(ghz + bz_ref[...])
    n = jnp.tanh(bin_ref[...] + r * (ghn + bhn_ref[...]))
    out_ref[...] = (1.0 - z) * n + z * h


def _gru_call(h128, wr, wz, wn, br, bz, bin_, bhn, *, interpret=False):
    BR = 2048
    grid = BATCH // BR
    in_spec = pl.BlockSpec((BR, 128), lambda i: (i, 0))
    out_spec = pl.BlockSpec((BR, MEMORY_DIM), lambda i: (i, 0))
    w_spec = pl.BlockSpec((MEMORY_DIM, MEMORY_DIM), lambda i: (0, 0))
    b_spec = pl.BlockSpec((1, MEMORY_DIM), lambda i: (0, 0))
    return pl.pallas_call(
        _gru_body,
        grid=(grid,),
        in_specs=[in_spec, w_spec, w_spec, w_spec,
                  b_spec, b_spec, b_spec, b_spec],
        out_specs=out_spec,
        out_shape=jax.ShapeDtypeStruct((BATCH, MEMORY_DIM), jnp.float32),
        interpret=interpret,
    )(h128, wr, wz, wn, br, bz, bin_, bhn)


def kernel(n_id, memory, W_ih, W_hh, b_ih, b_hh):
    del W_ih
    H = MEMORY_DIM
    memT = memory.T
    sid, spos = lax.sort_key_val(n_id, jnp.arange(BATCH, dtype=jnp.int32))
    h128 = _sc_gather()(memT, sid, spos)
    wr = W_hh[:H].T
    wz = W_hh[H:2 * H].T
    wn = W_hh[2 * H:].T
    br = (b_ih[:H] + b_hh[:H]).reshape(1, H)
    bz = (b_ih[H:2 * H] + b_hh[H:2 * H]).reshape(1, H)
    bin_ = b_ih[2 * H:].reshape(1, H)
    bhn = b_hh[2 * H:].reshape(1, H)
    new_mem = _gru_call(h128, wr, wz, wn, br, bz, bin_, bhn)
    last_update = jnp.zeros((n_id.shape[0],), dtype=jnp.int32)
    return (new_mem, last_update)


# speedup vs baseline: 4.1893x; 1.0784x over previous
import functools

import jax
import jax.numpy as jnp
from jax import lax
from jax.experimental import pallas as pl
from jax.experimental.pallas import tpu as pltpu
from jax.experimental.pallas import tpu_sc as plsc

NUM_NODES = 1000000
MEMORY_DIM = 64
BATCH = 16384
NC = 2
NS = 16
NW = NC * NS
B_PER_W = BATCH // NW

BLK_W = 512
BLK_SHIFT = 9
PAD_LANES = ((NUM_NODES + 127) // 128) * 128
BASE_MAX = PAD_LANES - BLK_W
NBUF = 3


def _sc_gather_body(memT, sid_hbm, spos_hbm, out_hbm,
                    ids_v, pos_v, blocks_v, rows_v, posb_v, dsem, ssem):
    wid = lax.axis_index("s") * NC + lax.axis_index("c")
    base = wid * B_PER_W
    pltpu.sync_copy(sid_hbm.at[pl.ds(base, B_PER_W)], ids_v)
    pltpu.sync_copy(spos_hbm.at[pl.ds(base, B_PER_W)], pos_v)

    def blk_copy(b):
        slot = lax.rem(b, NBUF)
        base = jnp.minimum(b * BLK_W, BASE_MAX)
        return pltpu.make_async_copy(
            memT.at[:, pl.ds(pl.multiple_of(base, 128), BLK_W)],
            blocks_v.at[slot],
            dsem.at[slot],
        )

    first = ids_v[pl.ds(0, 16)]
    lastv = ids_v[pl.ds(B_PER_W - 16, 16)]
    first_blk = lax.shift_right_logical(first[0], BLK_SHIFT)
    my_last_blk = lax.shift_right_logical(lastv[15], BLK_SHIFT)
    blk_copy(first_blk).start()
    blk_copy(jnp.minimum(first_blk + 1, my_last_blk)).start()

    def step_body(w, carry):
        blk_copy(jnp.minimum(w + 2, my_last_blk)).start()
        blk_copy(w).wait()
        return carry

    def group_body(g, waited):
        def chunk_body(cc, waited):
            c = g * 8 + cc
            v_ids = ids_v[pl.ds(c * 16, 16)]
            v_pos = pos_v[pl.ds(c * 16, 16)]
            posb_v[0, pl.ds(cc * 16, 16)] = v_pos
            for j in range(16):
                id_s = v_ids[j]
                tgt = lax.shift_right_logical(id_s, BLK_SHIFT)
                lax.fori_loop(waited + 1, tgt + 1, step_body, 0)
                waited = jnp.maximum(waited, tgt)

                slot = lax.rem(tgt, NBUF)
                col = id_s - jnp.minimum(tgt * BLK_W, BASE_MAX)
                lane_idx = jnp.broadcast_to(col, (16,))
                blk_ref = blocks_v.at[slot]
                for ccc in range(4):
                    ridx = jnp.arange(16, dtype=jnp.int32) + (16 * ccc)
                    val = plsc.load_gather(blk_ref, [ridx, lane_idx])
                    rows_v[cc * 16 + j, pl.ds(16 * ccc, 16)] = val
            return waited

        waited = lax.fori_loop(0, 8, chunk_body, waited)
        pltpu.async_copy(rows_v, out_hbm.at[posb_v.at[0]], ssem).wait()
        return waited

    lax.fori_loop(0, B_PER_W // 128, group_body, first_blk - 1)
    blk_copy(my_last_blk).wait()
    blk_copy(my_last_blk).wait()


@functools.cache
def _sc_gather():
    return pl.kernel(
        _sc_gather_body,
        out_type=jax.ShapeDtypeStruct((BATCH, 128), jnp.float32),
        mesh=plsc.VectorSubcoreMesh(core_axis_name="c", subcore_axis_name="s"),
        scratch_types=[
            pltpu.VMEM((B_PER_W,), jnp.int32),
            pltpu.VMEM((B_PER_W,), jnp.int32),
            pltpu.VMEM((NBUF, MEMORY_DIM, BLK_W), jnp.float32),
            pltpu.VMEM((128, 128), jnp.float32),
            pltpu.VMEM((1, 128), jnp.int32),
            pltpu.SemaphoreType.DMA((NBUF,)),
            pltpu.SemaphoreType.DMA,
        ],
        compiler_params=pltpu.CompilerParams(use_tc_tiling_on_sc=True, needs_layout_passes=False),
    )


def _gru_body(h_ref, wr_ref, wz_ref, wn_ref, br_ref, bz_ref, bin_ref,
              bhn_ref, out_ref):
    h = h_ref[...][:, :MEMORY_DIM]
    ghr = jnp.dot(h, wr_ref[...], preferred_element_type=jnp.float32)
    ghz = jnp.dot(h, wz_ref[...], preferred_element_type=jnp.float32)
    ghn = jnp.dot(h, wn_ref[...], preferred_element_type=jnp.float32)
    r = jax.nn.sigmoid(ghr + br_ref[...])
    z = jax.nn.sigmoid(ghz + bz_ref[...])
    n = jnp.tanh(bin_ref[...] + r * (ghn + bhn_ref[...]))
    out_ref[...] = (1.0 - z) * n + z * h


def _gru_call(h128, wr, wz, wn, br, bz, bin_, bhn, *, interpret=False):
    BR = 2048
    grid = BATCH // BR
    in_spec = pl.BlockSpec((BR, 128), lambda i: (i, 0))
    out_spec = pl.BlockSpec((BR, MEMORY_DIM), lambda i: (i, 0))
    w_spec = pl.BlockSpec((MEMORY_DIM, MEMORY_DIM), lambda i: (0, 0))
    b_spec = pl.BlockSpec((1, MEMORY_DIM), lambda i: (0, 0))
    return pl.pallas_call(
        _gru_body,
        grid=(grid,),
        in_specs=[in_spec, w_spec, w_spec, w_spec,
                  b_spec, b_spec, b_spec, b_spec],
        out_specs=out_spec,
        out_shape=jax.ShapeDtypeStruct((BATCH, MEMORY_DIM), jnp.float32),
        interpret=interpret,
    )(h128, wr, wz, wn, br, bz, bin_, bhn)


def kernel(n_id, memory, W_ih, W_hh, b_ih, b_hh):
    del W_ih
    H = MEMORY_DIM
    memT = memory.T
    sid, spos = lax.sort_key_val(n_id, jnp.arange(BATCH, dtype=jnp.int32))
    h128 = _sc_gather()(memT, sid, spos)
    wr = W_hh[:H].T
    wz = W_hh[H:2 * H].T
    wn = W_hh[2 * H:].T
    br = (b_ih[:H] + b_hh[:H]).reshape(1, H)
    bz = (b_ih[H:2 * H] + b_hh[H:2 * H]).reshape(1, H)
    bin_ = b_ih[2 * H:].reshape(1, H)
    bhn = b_hh[2 * H:].reshape(1, H)
    new_mem = _gru_call(h128, wr, wz, wn, br, bz, bin_, bhn)
    last_update = jnp.zeros((n_id.shape[0],), dtype=jnp.int32)
    return (new_mem, last_update)
